# baseline (speedup 1.0000x reference)
import functools

import jax
import jax.numpy as jnp
from jax import lax
from jax.experimental import pallas as pl
from jax.experimental.pallas import tpu as pltpu
from jax.experimental.pallas import tpu_sc as plsc

D = 128
B = 4096
NC = 2
NS = 16
NW = NC * NS
HB = B // 2
HPW = HB // NW

FIN_BLOCKS = 2
FTB = HB // FIN_BLOCKS

PART_BLOCKS = 4
PTB = B // PART_BLOCKS


def _gather_body(table_hbm, uid_hbm, out_hbm, idx_v, rows_v, sem):
    wid = lax.axis_index("s") * NC + lax.axis_index("c")
    base = wid * HPW
    pltpu.sync_copy(uid_hbm.at[pl.ds(base, HPW)], idx_v)
    pltpu.async_copy(table_hbm.at[idx_v], rows_v, sem).wait()
    pltpu.sync_copy(rows_v, out_hbm.at[pl.ds(base, HPW)])


@functools.cache
def _sc_gather():
    return pl.kernel(
        _gather_body,
        out_type=jax.ShapeDtypeStruct((HB, D), jnp.float32),
        mesh=plsc.VectorSubcoreMesh(core_axis_name="c", subcore_axis_name="s"),
        scratch_types=[
            pltpu.VMEM((HPW,), jnp.int32),
            pltpu.VMEM((HPW, D), jnp.float32),
            pltpu.SemaphoreType.DMA,
        ],
        compiler_params=pltpu.CompilerParams(skip_device_barrier=True),
    )


def _tc_partial_body(item_ref, w_ref, b_ref, out_ref):
    z = jnp.sum(item_ref[...] * w_ref[0:1, :], axis=1) + b_ref[0]
    out_ref[...] = z


def _tc_partial(item_emb, w_item, b1):
    return pl.pallas_call(
        _tc_partial_body,
        grid=(PART_BLOCKS,),
        in_specs=[
            pl.BlockSpec((PTB, D), lambda i: (i, 0)),
            pl.BlockSpec((1, D), lambda i: (0, 0)),
            pl.BlockSpec(memory_space=pltpu.SMEM),
        ],
        out_specs=pl.BlockSpec((PTB,), lambda i: (i,)),
        out_shape=jax.ShapeDtypeStruct((B,), jnp.float32),
        compiler_params=pltpu.CompilerParams(
            dimension_semantics=("arbitrary",)),
    )(item_emb, w_item, b1)


def _tc_final_body(user_ref, part_ref, w_ref, out_ref):
    z = jnp.sum(user_ref[...] * w_ref[0:1, :], axis=1) + part_ref[...]
    out_ref[...] = 1.0 / (1.0 + jnp.exp(-z))


def _tc_final(user_embs, partial, w_user, half):
    return pl.pallas_call(
        _tc_final_body,
        grid=(FIN_BLOCKS,),
        in_specs=[
            pl.BlockSpec((FTB, D), lambda i: (i, 0)),
            pl.BlockSpec((FTB,), lambda i, h=half: (h * FIN_BLOCKS + i,)),
            pl.BlockSpec((1, D), lambda i: (0, 0)),
        ],
        out_specs=pl.BlockSpec((FTB,), lambda i: (i,)),
        out_shape=jax.ShapeDtypeStruct((HB,), jnp.float32),
        compiler_params=pltpu.CompilerParams(
            dimension_semantics=("arbitrary",)),
    )(user_embs, partial, w_user)


def kernel(user_id, item_emb, user_table, W, b):
    uid = user_id.astype(jnp.int32)
    w2 = W.reshape(2, D)
    ue0 = _sc_gather()(user_table, lax.slice(uid, (0,), (HB,)))
    ue1 = _sc_gather()(user_table, lax.slice(uid, (HB,), (B,)))
    partial = _tc_partial(item_emb, w2[1:2], b)
    o0 = _tc_final(ue0, partial, w2[0:1], 0)
    o1 = _tc_final(ue1, partial, w2[0:1], 1)
    return jnp.concatenate([o0, o1]).reshape(B, 1)
